# Initial kernel scaffold; baseline (speedup 1.0000x reference)
import jax
import jax.numpy as jnp
from jax.experimental import pallas as pl


def kernel(x, edge_attr, edge_index, batch, W1, b1, gn1_w, gn1_b, gn1_ms, W2, b2, gn2_w, gn2_b, gn2_ms, gW1, gb1, gbn_g, gbn_b, gW2, gb2, lW1, lb1, lbn1_g, lbn1_b, lW2, lb2, lbn2_g, lbn2_b, lW3, lb3):
    raise NotImplementedError("write your pallas kernel here")


# speedup vs baseline: 5.9181x; 5.9181x over previous
import functools
import jax
import jax.numpy as jnp
from jax import lax
from jax.experimental import pallas as pl
from jax.experimental.pallas import tpu as pltpu, tpu_sc as plsc

N = 10000
E = 320000
D = 128
G = 64
LIN = 256
EPS = 1e-5

NP_ = 10240
NB = 128
NBLK = NP_ // NB
NC = 2
NS = 16
NW = NC * NS
C = 128
EW_PER = 10240
KC = EW_PER // C
CA = 64
KA = EW_PER // CA
EP = NW * EW_PER
ROWS_PER_TILE = NP_ // NS
NAGG = 10112
AGG_ROWS = NAGG // NS


def _sc_deg_body(col_hbm, ew_hbm, out_hbm, colbuf, ewc, ewb, zbuf, acc_sh, sem):
    cid = lax.axis_index("c")
    sid = lax.axis_index("s")
    wid = cid * NS + sid

    def zrow(r, _):
        for g in range(D // 16):
            zbuf[r, pl.ds(g * 16, 16)] = jnp.zeros((16,), jnp.float32)
        return 0
    lax.fori_loop(0, C, zrow, 0)
    base = sid * ROWS_PER_TILE
    for z in range(ROWS_PER_TILE // C):
        pltpu.sync_copy(zbuf, acc_sh.at[pl.ds(base + z * C, C)])
    plsc.subcore_barrier()

    def chunk(j, _):
        pltpu.sync_copy(col_hbm.at[wid, j], colbuf)
        pltpu.sync_copy(ew_hbm.at[wid, j], ewc)

        def fill(i2, _):
            ev = ewc[pl.ds(i2 * 16, 16)]
            for k in range(16):
                for g in range(D // 16):
                    ewb[i2 * 16 + k, pl.ds(g * 16, 16)] = jnp.zeros((16,), jnp.float32) + ev[k]
            return 0
        lax.fori_loop(0, C // 16, fill, 0)
        pltpu.sync_copy(ewb, acc_sh.at[colbuf], add=True)
        return 0
    lax.fori_loop(0, KC, chunk, 0)
    plsc.subcore_barrier()

    for z in range(ROWS_PER_TILE // C):
        pltpu.sync_copy(acc_sh.at[pl.ds(base + z * C, C)], zbuf)
        pltpu.sync_copy(zbuf, out_hbm.at[cid, pl.ds(base + z * C, C)])


def _sc_agg_body(rce_hbm, hp_hbm, out_hbm, stg, buf, acc_sh, sem):
    cid = lax.axis_index("c")
    sid = lax.axis_index("s")
    wid = cid * NS + sid

    def zrow(r, _):
        for g in range(D // 16):
            buf[r, pl.ds(g * 16, 16)] = jnp.zeros((16,), jnp.float32)
        return 0
    lax.fori_loop(0, C, zrow, 0)
    base = sid * ROWS_PER_TILE
    for z in range(ROWS_PER_TILE // C):
        pltpu.sync_copy(buf, acc_sh.at[pl.ds(base + z * C, C)])
    plsc.subcore_barrier()

    def chunk(j, _):
        pltpu.sync_copy(rce_hbm.at[wid, j], stg)
        pltpu.async_copy(hp_hbm.at[stg.at[0]], buf, sem).wait()

        def scale(i2, _):
            ev = plsc.bitcast(stg[2, pl.ds(i2 * 16, 16)], jnp.float32)
            for k in range(16):
                s = ev[k]
                r = i2 * 16 + k
                for g in range(D // 16):
                    buf[r, pl.ds(g * 16, 16)] = buf[r, pl.ds(g * 16, 16)] * s
            return 0
        lax.fori_loop(0, C // 16, scale, 0)
        pltpu.sync_copy(buf, acc_sh.at[stg.at[1]], add=True)
        return 0
    lax.fori_loop(0, KC, chunk, 0)
    plsc.subcore_barrier()

    for z in range(ROWS_PER_TILE // C):
        pltpu.sync_copy(acc_sh.at[pl.ds(base + z * C, C)], buf)
        pltpu.sync_copy(buf, out_hbm.at[cid, pl.ds(base + z * C, C)])


def _make_sc_deg():
    mesh = plsc.VectorSubcoreMesh(core_axis_name="c", subcore_axis_name="s",
                                  num_cores=NC, num_subcores=NS)
    return pl.kernel(
        _sc_deg_body,
        out_type=jax.ShapeDtypeStruct((NC, NP_, D), jnp.float32),
        mesh=mesh,
        compiler_params=pltpu.CompilerParams(needs_layout_passes=False),
        scratch_types=[
            pltpu.VMEM((C,), jnp.int32),
            pltpu.VMEM((C,), jnp.float32),
            pltpu.VMEM((C, D), jnp.float32),
            pltpu.VMEM((C, D), jnp.float32),
            pltpu.VMEM_SHARED((NP_, D), jnp.float32),
            pltpu.SemaphoreType.DMA,
        ],
    )


def _make_sc_agg():
    mesh = plsc.VectorSubcoreMesh(core_axis_name="c", subcore_axis_name="s",
                                  num_cores=NC, num_subcores=NS)
    return pl.kernel(
        _sc_agg_body,
        out_type=jax.ShapeDtypeStruct((NC, NP_, D), jnp.float32),
        mesh=mesh,
        compiler_params=pltpu.CompilerParams(needs_layout_passes=False),
        scratch_types=[
            pltpu.VMEM((3, C), jnp.int32),
            pltpu.VMEM((C, D), jnp.float32),
            pltpu.VMEM_SHARED((NP_, D), jnp.float32),
            pltpu.SemaphoreType.DMA,
        ],
    )


def _rows_valid(i):
    ridx = i * NB + lax.broadcasted_iota(jnp.int32, (NB, 1), 0)
    return ridx < N


def _onehot(batch_blk, valid):
    b = jnp.transpose(batch_blk, (1, 0))
    gi = lax.broadcasted_iota(jnp.int32, (1, G), 1)
    return jnp.where((b == gi) & valid, 1.0, 0.0).astype(jnp.float32)


def _tc_prep_kernel(x_ref, w_ref, d0_ref, d1_ref, batch_ref,
                    hp_ref, dinv_ref, cnt_ref):
    i = pl.program_id(0)
    deg = 1.0 + d0_ref[:, 0:1] + d1_ref[:, 0:1]
    dinv = jnp.where(deg > 0, lax.rsqrt(jnp.maximum(deg, 1e-12)), 0.0)
    hp_ref[...] = dinv * jnp.dot(x_ref[...], w_ref[...],
                                 preferred_element_type=jnp.float32)
    dinv_ref[...] = jnp.transpose(dinv, (1, 0))
    valid = _rows_valid(i)
    oh = _onehot(batch_ref[...], valid)
    c = jnp.sum(oh, axis=0, keepdims=True)
    @pl.when(i == 0)
    def _():
        cnt_ref[...] = jnp.zeros_like(cnt_ref)
    cnt_ref[...] += c


def _tc_prep(x, w, d0, d1, batch2):
    return pl.pallas_call(
        _tc_prep_kernel,
        grid=(NBLK,),
        in_specs=[
            pl.BlockSpec((NB, D), lambda i: (i, 0)),
            pl.BlockSpec((D, D), lambda i: (0, 0)),
            pl.BlockSpec((NB, D), lambda i: (i, 0)),
            pl.BlockSpec((NB, D), lambda i: (i, 0)),
            pl.BlockSpec((1, NB), lambda i: (0, i)),
        ],
        out_specs=[
            pl.BlockSpec((NB, D), lambda i: (i, 0)),
            pl.BlockSpec((1, NB), lambda i: (0, i)),
            pl.BlockSpec((1, G), lambda i: (0, 0)),
        ],
        out_shape=[
            jax.ShapeDtypeStruct((NP_, D), jnp.float32),
            jax.ShapeDtypeStruct((1, NP_), jnp.float32),
            jax.ShapeDtypeStruct((1, G), jnp.float32),
        ],
    )(x, w, d0, d1, batch2)


def _tc_stats_kernel(a0_ref, a1_ref, hp_ref, dinv_ref, b_ref, batch_ref,
                     y_ref, s1_ref, s2_ref):
    i = pl.program_id(0)
    dv = jnp.transpose(dinv_ref[...], (1, 0))
    valid = _rows_valid(i)
    y = dv * (a0_ref[...] + a1_ref[...] + hp_ref[...]) + b_ref[...]
    y = jnp.where(valid, y, 0.0)
    y_ref[...] = y
    oh = _onehot(batch_ref[...], valid)
    ohT = jnp.transpose(oh, (1, 0))
    @pl.when(i == 0)
    def _():
        s1_ref[...] = jnp.zeros_like(s1_ref)
        s2_ref[...] = jnp.zeros_like(s2_ref)
    s1_ref[...] += jnp.dot(ohT, y, preferred_element_type=jnp.float32, precision=lax.Precision.HIGHEST)
    s2_ref[...] += jnp.dot(ohT, y * y, preferred_element_type=jnp.float32, precision=lax.Precision.HIGHEST)


def _tc_stats(a0, a1, hp, dinv, bias, batch2):
    return pl.pallas_call(
        _tc_stats_kernel,
        grid=(NBLK,),
        in_specs=[
            pl.BlockSpec((NB, D), lambda i: (i, 0)),
            pl.BlockSpec((NB, D), lambda i: (i, 0)),
            pl.BlockSpec((NB, D), lambda i: (i, 0)),
            pl.BlockSpec((1, NB), lambda i: (0, i)),
            pl.BlockSpec((1, D), lambda i: (0, 0)),
            pl.BlockSpec((1, NB), lambda i: (0, i)),
        ],
        out_specs=[
            pl.BlockSpec((NB, D), lambda i: (i, 0)),
            pl.BlockSpec((G, D), lambda i: (0, 0)),
            pl.BlockSpec((G, D), lambda i: (0, 0)),
        ],
        out_shape=[
            jax.ShapeDtypeStruct((NP_, D), jnp.float32),
            jax.ShapeDtypeStruct((G, D), jnp.float32),
            jax.ShapeDtypeStruct((G, D), jnp.float32),
        ],
    )(a0, a1, hp, dinv, bias, batch2)


def _gn_apply(y, batch_blk, valid, s1, s2, cnt, w, b, ms):
    cntc = jnp.maximum(jnp.transpose(cnt, (1, 0)), 1.0)
    mean = s1 / cntc
    m2 = s2 / cntc
    var = m2 - (2.0 * ms - ms * ms) * mean * mean
    oh = _onehot(batch_blk, valid)
    mean_r = jnp.dot(oh, mean, preferred_element_type=jnp.float32, precision=lax.Precision.HIGHEST)
    var_r = jnp.dot(oh, var, preferred_element_type=jnp.float32, precision=lax.Precision.HIGHEST)
    out = y - mean_r * ms
    return w * out * lax.rsqrt(var_r + EPS) + b


def _tc_gn1_kernel(y_ref, s1_ref, s2_ref, cnt_ref, w_ref, b_ref, ms_ref,
                   batch_ref, dinv_ref, w2_ref, hp2_ref):
    i = pl.program_id(0)
    valid = _rows_valid(i)
    z = _gn_apply(y_ref[...], batch_ref[...], valid, s1_ref[...], s2_ref[...],
                  cnt_ref[...], w_ref[...], b_ref[...], ms_ref[...])
    z = jnp.maximum(z, 0.0)
    dv = jnp.transpose(dinv_ref[...], (1, 0))
    hp2_ref[...] = dv * jnp.dot(z, w2_ref[...],
                                preferred_element_type=jnp.float32)


def _tc_gn1(y, s1, s2, cnt, w, b, ms, batch2, dinv, w2):
    return pl.pallas_call(
        _tc_gn1_kernel,
        grid=(NBLK,),
        in_specs=[
            pl.BlockSpec((NB, D), lambda i: (i, 0)),
            pl.BlockSpec((G, D), lambda i: (0, 0)),
            pl.BlockSpec((G, D), lambda i: (0, 0)),
            pl.BlockSpec((1, G), lambda i: (0, 0)),
            pl.BlockSpec((1, D), lambda i: (0, 0)),
            pl.BlockSpec((1, D), lambda i: (0, 0)),
            pl.BlockSpec((1, D), lambda i: (0, 0)),
            pl.BlockSpec((1, NB), lambda i: (0, i)),
            pl.BlockSpec((1, NB), lambda i: (0, i)),
            pl.BlockSpec((D, D), lambda i: (0, 0)),
        ],
        out_specs=pl.BlockSpec((NB, D), lambda i: (i, 0)),
        out_shape=jax.ShapeDtypeStruct((NP_, D), jnp.float32),
    )(y, s1, s2, cnt, w, b, ms, batch2, dinv, w2)


def _tc_gn2_kernel(y_ref, s1_ref, s2_ref, cnt_ref, w_ref, b_ref, ms_ref,
                   batch_ref, gw1_ref, gb1_ref,
                   h_ref, t_ref, st1_ref, st2_ref):
    i = pl.program_id(0)
    valid = _rows_valid(i)
    z = _gn_apply(y_ref[...], batch_ref[...], valid, s1_ref[...], s2_ref[...],
                  cnt_ref[...], w_ref[...], b_ref[...], ms_ref[...])
    h = jnp.maximum(z, 0.0)
    h_ref[...] = h
    t = jnp.dot(h, gw1_ref[...], preferred_element_type=jnp.float32) + gb1_ref[...]
    t_ref[...] = t
    tm = jnp.where(valid, t, 0.0)
    @pl.when(i == 0)
    def _():
        st1_ref[...] = jnp.zeros_like(st1_ref)
        st2_ref[...] = jnp.zeros_like(st2_ref)
    st1_ref[...] += jnp.sum(tm, axis=0, keepdims=True)
    st2_ref[...] += jnp.sum(tm * tm, axis=0, keepdims=True)


def _tc_gn2(y, s1, s2, cnt, w, b, ms, batch2, gw1, gb1):
    return pl.pallas_call(
        _tc_gn2_kernel,
        grid=(NBLK,),
        in_specs=[
            pl.BlockSpec((NB, D), lambda i: (i, 0)),
            pl.BlockSpec((G, D), lambda i: (0, 0)),
            pl.BlockSpec((G, D), lambda i: (0, 0)),
            pl.BlockSpec((1, G), lambda i: (0, 0)),
            pl.BlockSpec((1, D), lambda i: (0, 0)),
            pl.BlockSpec((1, D), lambda i: (0, 0)),
            pl.BlockSpec((1, D), lambda i: (0, 0)),
            pl.BlockSpec((1, NB), lambda i: (0, i)),
            pl.BlockSpec((D, 2 * D), lambda i: (0, 0)),
            pl.BlockSpec((1, 2 * D), lambda i: (0, 0)),
        ],
        out_specs=[
            pl.BlockSpec((NB, D), lambda i: (i, 0)),
            pl.BlockSpec((NB, 2 * D), lambda i: (i, 0)),
            pl.BlockSpec((1, 2 * D), lambda i: (0, 0)),
            pl.BlockSpec((1, 2 * D), lambda i: (0, 0)),
        ],
        out_shape=[
            jax.ShapeDtypeStruct((NP_, D), jnp.float32),
            jax.ShapeDtypeStruct((NP_, 2 * D), jnp.float32),
            jax.ShapeDtypeStruct((1, 2 * D), jnp.float32),
            jax.ShapeDtypeStruct((1, 2 * D), jnp.float32),
        ],
    )(y, s1, s2, cnt, w, b, ms, batch2, gw1, gb1)


def _tc_gate_kernel(t_ref, st1_ref, st2_ref, g_ref, b_ref, gw2_ref, gb2_ref,
                    batch_ref, gate_ref, gmax_ref):
    i = pl.program_id(0)
    valid = _rows_valid(i)
    m = st1_ref[...] / float(N)
    v = st2_ref[...] / float(N) - m * m
    tb = g_ref[...] * (t_ref[...] - m) * lax.rsqrt(v + EPS) + b_ref[...]
    tb = jnp.maximum(tb, 0.0)
    gate = jnp.dot(tb, gw2_ref[...], preferred_element_type=jnp.float32) \
        + gb2_ref[...]
    gate_ref[...] = jnp.transpose(gate, (1, 0))
    oh = _onehot(batch_ref[...], valid)
    gm = jnp.max(jnp.where(oh > 0, gate, -3e38), axis=0, keepdims=True)
    @pl.when(i == 0)
    def _():
        gmax_ref[...] = jnp.full_like(gmax_ref, -3e38)
    gmax_ref[...] = jnp.maximum(gmax_ref[...], gm)


def _tc_gate(t, st1, st2, g, b, gw2, gb2, batch2):
    return pl.pallas_call(
        _tc_gate_kernel,
        grid=(NBLK,),
        in_specs=[
            pl.BlockSpec((NB, 2 * D), lambda i: (i, 0)),
            pl.BlockSpec((1, 2 * D), lambda i: (0, 0)),
            pl.BlockSpec((1, 2 * D), lambda i: (0, 0)),
            pl.BlockSpec((1, 2 * D), lambda i: (0, 0)),
            pl.BlockSpec((1, 2 * D), lambda i: (0, 0)),
            pl.BlockSpec((2 * D, 1), lambda i: (0, 0)),
            pl.BlockSpec((1, 1), lambda i: (0, 0)),
            pl.BlockSpec((1, NB), lambda i: (0, i)),
        ],
        out_specs=[
            pl.BlockSpec((1, NB), lambda i: (0, i)),
            pl.BlockSpec((1, G), lambda i: (0, 0)),
        ],
        out_shape=[
            jax.ShapeDtypeStruct((1, NP_), jnp.float32),
            jax.ShapeDtypeStruct((1, G), jnp.float32),
        ],
    )(t, st1, st2, g, b, gw2, gb2, batch2)


def _tc_pool_kernel(gate_ref, gmax_ref, h_ref, batch_ref, num_ref, den_ref):
    i = pl.program_id(0)
    valid = _rows_valid(i)
    oh = _onehot(batch_ref[...], valid)
    gmax_r = jnp.dot(oh, jnp.transpose(gmax_ref[...], (1, 0)),
                     preferred_element_type=jnp.float32, precision=lax.Precision.HIGHEST)
    gate = jnp.transpose(gate_ref[...], (1, 0))
    e = jnp.where(valid, jnp.exp(gate - gmax_r), 0.0)
    ohT = jnp.transpose(oh, (1, 0))
    @pl.when(i == 0)
    def _():
        num_ref[...] = jnp.zeros_like(num_ref)
        den_ref[...] = jnp.zeros_like(den_ref)
    num_ref[...] += jnp.dot(ohT, e * h_ref[...],
                            preferred_element_type=jnp.float32, precision=lax.Precision.HIGHEST)
    den_ref[...] += jnp.sum(oh * e, axis=0, keepdims=True)


def _tc_pool(gate, gmax, h, batch2):
    return pl.pallas_call(
        _tc_pool_kernel,
        grid=(NBLK,),
        in_specs=[
            pl.BlockSpec((1, NB), lambda i: (0, i)),
            pl.BlockSpec((1, G), lambda i: (0, 0)),
            pl.BlockSpec((NB, D), lambda i: (i, 0)),
            pl.BlockSpec((1, NB), lambda i: (0, i)),
        ],
        out_specs=[
            pl.BlockSpec((G, D), lambda i: (0, 0)),
            pl.BlockSpec((1, G), lambda i: (0, 0)),
        ],
        out_shape=[
            jax.ShapeDtypeStruct((G, D), jnp.float32),
            jax.ShapeDtypeStruct((1, G), jnp.float32),
        ],
    )(gate, gmax, h, batch2)


def _bn_rows(x, g, b):
    m = jnp.mean(x, axis=0, keepdims=True)
    v = jnp.mean(x * x, axis=0, keepdims=True) - m * m
    return g * (x - m) * lax.rsqrt(v + EPS) + b


def _tc_head_kernel(num_ref, den_ref, lw1_ref, lb1_ref, g1_ref, bb1_ref,
                    lw2_ref, lb2_ref, g2_ref, bb2_ref, lw3_ref, lb3_ref,
                    out_ref):
    den = jnp.maximum(jnp.transpose(den_ref[...], (1, 0)), 1e-16)
    pooled = num_ref[...] / den
    z = jnp.dot(pooled, lw1_ref[...], preferred_element_type=jnp.float32) \
        + lb1_ref[...]
    z = jnp.maximum(_bn_rows(z, g1_ref[...], bb1_ref[...]), 0.0)
    z = jnp.dot(z, lw2_ref[...], preferred_element_type=jnp.float32) \
        + lb2_ref[...]
    z = jnp.maximum(_bn_rows(z, g2_ref[...], bb2_ref[...]), 0.0)
    out_ref[...] = jnp.dot(z, lw3_ref[...],
                           preferred_element_type=jnp.float32) + lb3_ref[...]


def _tc_head(num, den, lw1, lb1, g1, bb1, lw2, lb2, g2, bb2, lw3, lb3):
    specs = [
        pl.BlockSpec((G, D), lambda: (0, 0)),
        pl.BlockSpec((1, G), lambda: (0, 0)),
        pl.BlockSpec((D, LIN), lambda: (0, 0)),
        pl.BlockSpec((1, LIN), lambda: (0, 0)),
        pl.BlockSpec((1, LIN), lambda: (0, 0)),
        pl.BlockSpec((1, LIN), lambda: (0, 0)),
        pl.BlockSpec((LIN, LIN), lambda: (0, 0)),
        pl.BlockSpec((1, LIN), lambda: (0, 0)),
        pl.BlockSpec((1, LIN), lambda: (0, 0)),
        pl.BlockSpec((1, LIN), lambda: (0, 0)),
        pl.BlockSpec((LIN, 1), lambda: (0, 0)),
        pl.BlockSpec((1, 1), lambda: (0, 0)),
    ]
    return pl.pallas_call(
        _tc_head_kernel,
        in_specs=specs,
        out_specs=pl.BlockSpec((G, 1), lambda: (0, 0)),
        out_shape=jax.ShapeDtypeStruct((G, 1), jnp.float32),
    )(num, den, lw1, lb1, g1, bb1, lw2, lb2, g2, bb2, lw3, lb3)


def kernel(x, edge_attr, edge_index, batch,
           W1, b1, gn1_w, gn1_b, gn1_ms,
           W2, b2, gn2_w, gn2_b, gn2_ms,
           gW1, gb1, gbn_g, gbn_b, gW2, gb2,
           lW1, lb1, lbn1_g, lbn1_b, lW2, lb2, lbn2_g, lbn2_b, lW3, lb3):
    f32 = jnp.float32
    row = edge_index[0].astype(jnp.int32)
    col = edge_index[1].astype(jnp.int32)
    ew = edge_attr.astype(f32)

    pad = EP - E
    rowf = jnp.concatenate([row, jnp.zeros((pad,), jnp.int32)])
    colf = jnp.concatenate([col, jnp.zeros((pad,), jnp.int32)])
    ewf = jnp.concatenate([ew, jnp.zeros((pad,), f32)])
    colp = colf.reshape(NW, KC, C)
    ewp = ewf.reshape(NW, KC, C)
    ewbits = lax.bitcast_convert_type(ewf, jnp.int32)
    rce = jnp.stack([rowf.reshape(NW, KC, C), colf.reshape(NW, KC, C),
                     ewbits.reshape(NW, KC, C)], axis=2)

    xp = jnp.pad(x.astype(f32), ((0, NP_ - N), (0, 0)))
    batchp = jnp.pad(batch.astype(jnp.int32), (0, NP_ - N),
                     constant_values=G).reshape(1, NP_)

    r2 = lambda a: a.astype(f32).reshape(1, -1)

    sc_deg = _make_sc_deg()

    sc_agg = _make_sc_agg()

    dacc = sc_deg(colp, ewp)
    hp1, dinv, cnt = _tc_prep(xp, W1.astype(f32), dacc[0], dacc[1], batchp)

    a1 = sc_agg(rce, hp1)
    y1, s1, s2 = _tc_stats(a1[0], a1[1], hp1, dinv, r2(b1), batchp)
    hp2 = _tc_gn1(y1, s1, s2, cnt, r2(gn1_w), r2(gn1_b), r2(gn1_ms),
                  batchp, dinv, W2.astype(f32))

    a2 = sc_agg(rce, hp2)
    y2, s1b, s2b = _tc_stats(a2[0], a2[1], hp2, dinv, r2(b2), batchp)
    h, t, st1, st2 = _tc_gn2(y2, s1b, s2b, cnt, r2(gn2_w), r2(gn2_b),
                             r2(gn2_ms), batchp, gW1.astype(f32), r2(gb1))

    gate, gmax = _tc_gate(t, st1, st2, r2(gbn_g), r2(gbn_b),
                          gW2.astype(f32), gb2.reshape(1, 1).astype(f32),
                          batchp)
    num, den = _tc_pool(gate, gmax, h, batchp)

    out = _tc_head(num, den, lW1.astype(f32), r2(lb1), r2(lbn1_g), r2(lbn1_b),
                   lW2.astype(f32), r2(lb2), r2(lbn2_g), r2(lbn2_b),
                   lW3.astype(f32), lb3.reshape(1, 1).astype(f32))
    return out


# speedup vs baseline: 6.0353x; 1.0198x over previous
import functools
import jax
import jax.numpy as jnp
from jax import lax
from jax.experimental import pallas as pl
from jax.experimental.pallas import tpu as pltpu, tpu_sc as plsc

N = 10000
E = 320000
D = 128
G = 64
LIN = 256
EPS = 1e-5

NP_ = 10240
NB = 128
NBLK = NP_ // NB
NC = 2
NS = 16
NW = NC * NS
C = 128
EW_PER = 10240
KC = EW_PER // C
CA = 64
KA = EW_PER // CA
EP = NW * EW_PER
ROWS_PER_TILE = NP_ // NS
KH = KC // 2
NAGG = 10112
AGG_ROWS = NAGG // NS


def _sc_deg_body(rce_hbm, out_hbm, stg, ewb, acc_sh, sem):
    cid = lax.axis_index("c")
    sid = lax.axis_index("s")
    wid = cid * NS + sid

    def zrow(r, _):
        for g in range(D // 16):
            ewb[r, pl.ds(g * 16, 16)] = jnp.zeros((16,), jnp.float32)
        return 0
    lax.fori_loop(0, C, zrow, 0)
    base = sid * ROWS_PER_TILE
    for z in range(ROWS_PER_TILE // C):
        pltpu.sync_copy(ewb, acc_sh.at[pl.ds(base + z * C, C)])
    plsc.subcore_barrier()

    lane = lax.iota(jnp.int32, 16)

    def chunk(jh, _):
        pltpu.sync_copy(rce_hbm.at[wid, jh], stg)
        for h in range(2):
            def fill(i2, _):
                ev = plsc.bitcast(stg[4 + h, pl.ds(i2 * 16, 16)], jnp.float32)
                for k in range(16):
                    ewb[i2 * 16 + k, pl.ds(0, 16)] = jnp.where(
                        lane == k, ev[k], 0.0)
                return 0
            lax.fori_loop(0, C // 16, fill, 0)
            pltpu.sync_copy(ewb, acc_sh.at[stg.at[2 + h]], add=True)
        return 0
    lax.fori_loop(0, KH, chunk, 0)
    plsc.subcore_barrier()

    for z in range(ROWS_PER_TILE // C):
        pltpu.sync_copy(acc_sh.at[pl.ds(base + z * C, C)], ewb)
        pltpu.sync_copy(ewb, out_hbm.at[cid, pl.ds(base + z * C, C)])


def _sc_agg_body(rce_hbm, hp_hbm, out_hbm, stg, buf, acc_sh, sem):
    cid = lax.axis_index("c")
    sid = lax.axis_index("s")
    wid = cid * NS + sid

    def zrow(r, _):
        for g in range(D // 16):
            buf[r, pl.ds(g * 16, 16)] = jnp.zeros((16,), jnp.float32)
        return 0
    lax.fori_loop(0, C, zrow, 0)
    base = sid * ROWS_PER_TILE
    for z in range(ROWS_PER_TILE // C):
        pltpu.sync_copy(buf, acc_sh.at[pl.ds(base + z * C, C)])
    plsc.subcore_barrier()

    def chunk(jh, _):
        pltpu.sync_copy(rce_hbm.at[wid, jh], stg)
        for h in range(2):
            pltpu.async_copy(hp_hbm.at[stg.at[h]], buf, sem).wait()

            def scale(i2, _):
                ev = plsc.bitcast(stg[4 + h, pl.ds(i2 * 16, 16)], jnp.float32)
                for k in range(16):
                    sc = ev[k]
                    r = i2 * 16 + k
                    for g in range(D // 16):
                        buf[r, pl.ds(g * 16, 16)] = (
                            buf[r, pl.ds(g * 16, 16)] * sc)
                return 0
            lax.fori_loop(0, C // 16, scale, 0)
            pltpu.sync_copy(buf, acc_sh.at[stg.at[2 + h]], add=True)
        return 0
    lax.fori_loop(0, KH, chunk, 0)
    plsc.subcore_barrier()

    for z in range(ROWS_PER_TILE // C):
        pltpu.sync_copy(acc_sh.at[pl.ds(base + z * C, C)], buf)
        pltpu.sync_copy(buf, out_hbm.at[cid, pl.ds(base + z * C, C)])


def _make_sc_deg():
    mesh = plsc.VectorSubcoreMesh(core_axis_name="c", subcore_axis_name="s",
                                  num_cores=NC, num_subcores=NS)
    return pl.kernel(
        _sc_deg_body,
        out_type=jax.ShapeDtypeStruct((NC, NP_, D), jnp.float32),
        mesh=mesh,
        compiler_params=pltpu.CompilerParams(needs_layout_passes=False),
        scratch_types=[
            pltpu.VMEM((6, C), jnp.int32),
            pltpu.VMEM((C, D), jnp.float32),
            pltpu.VMEM_SHARED((NP_, D), jnp.float32),
            pltpu.SemaphoreType.DMA,
        ],
    )


def _make_sc_agg():
    mesh = plsc.VectorSubcoreMesh(core_axis_name="c", subcore_axis_name="s",
                                  num_cores=NC, num_subcores=NS)
    return pl.kernel(
        _sc_agg_body,
        out_type=jax.ShapeDtypeStruct((NC, NP_, D), jnp.float32),
        mesh=mesh,
        compiler_params=pltpu.CompilerParams(needs_layout_passes=False),
        scratch_types=[
            pltpu.VMEM((6, C), jnp.int32),
            pltpu.VMEM((C, D), jnp.float32),
            pltpu.VMEM_SHARED((NP_, D), jnp.float32),
            pltpu.SemaphoreType.DMA,
        ],
    )


def _rows_valid(i):
    ridx = i * NB + lax.broadcasted_iota(jnp.int32, (NB, 1), 0)
    return ridx < N


def _onehot(batch_blk, valid):
    b = jnp.transpose(batch_blk, (1, 0))
    gi = lax.broadcasted_iota(jnp.int32, (1, G), 1)
    return jnp.where((b == gi) & valid, 1.0, 0.0).astype(jnp.float32)


def _tc_prep_kernel(x_ref, w_ref, d0_ref, d1_ref, batch_ref,
                    hp_ref, dinv_ref, cnt_ref):
    i = pl.program_id(0)
    deg = 1.0 + jnp.sum(d0_ref[...] + d1_ref[...], axis=1, keepdims=True)
    dinv = jnp.where(deg > 0, lax.rsqrt(jnp.maximum(deg, 1e-12)), 0.0)
    hp_ref[...] = dinv * jnp.dot(x_ref[...], w_ref[...],
                                 preferred_element_type=jnp.float32)
    dinv_ref[...] = jnp.transpose(dinv, (1, 0))
    valid = _rows_valid(i)
    oh = _onehot(batch_ref[...], valid)
    c = jnp.sum(oh, axis=0, keepdims=True)
    @pl.when(i == 0)
    def _():
        cnt_ref[...] = jnp.zeros_like(cnt_ref)
    cnt_ref[...] += c


def _tc_prep(x, w, d0, d1, batch2):
    return pl.pallas_call(
        _tc_prep_kernel,
        grid=(NBLK,),
        in_specs=[
            pl.BlockSpec((NB, D), lambda i: (i, 0)),
            pl.BlockSpec((D, D), lambda i: (0, 0)),
            pl.BlockSpec((NB, D), lambda i: (i, 0)),
            pl.BlockSpec((NB, D), lambda i: (i, 0)),
            pl.BlockSpec((1, NB), lambda i: (0, i)),
        ],
        out_specs=[
            pl.BlockSpec((NB, D), lambda i: (i, 0)),
            pl.BlockSpec((1, NB), lambda i: (0, i)),
            pl.BlockSpec((1, G), lambda i: (0, 0)),
        ],
        out_shape=[
            jax.ShapeDtypeStruct((NP_, D), jnp.float32),
            jax.ShapeDtypeStruct((1, NP_), jnp.float32),
            jax.ShapeDtypeStruct((1, G), jnp.float32),
        ],
    )(x, w, d0, d1, batch2)


def _tc_stats_kernel(a0_ref, a1_ref, hp_ref, dinv_ref, b_ref, batch_ref,
                     y_ref, s1_ref, s2_ref):
    i = pl.program_id(0)
    dv = jnp.transpose(dinv_ref[...], (1, 0))
    valid = _rows_valid(i)
    y = dv * (a0_ref[...] + a1_ref[...] + hp_ref[...]) + b_ref[...]
    y = jnp.where(valid, y, 0.0)
    y_ref[...] = y
    oh = _onehot(batch_ref[...], valid)
    ohT = jnp.transpose(oh, (1, 0))
    @pl.when(i == 0)
    def _():
        s1_ref[...] = jnp.zeros_like(s1_ref)
        s2_ref[...] = jnp.zeros_like(s2_ref)
    s1_ref[...] += jnp.dot(ohT, y, preferred_element_type=jnp.float32, precision=lax.Precision.HIGHEST)
    s2_ref[...] += jnp.dot(ohT, y * y, preferred_element_type=jnp.float32, precision=lax.Precision.HIGHEST)


def _tc_stats(a0, a1, hp, dinv, bias, batch2):
    return pl.pallas_call(
        _tc_stats_kernel,
        grid=(NBLK,),
        in_specs=[
            pl.BlockSpec((NB, D), lambda i: (i, 0)),
            pl.BlockSpec((NB, D), lambda i: (i, 0)),
            pl.BlockSpec((NB, D), lambda i: (i, 0)),
            pl.BlockSpec((1, NB), lambda i: (0, i)),
            pl.BlockSpec((1, D), lambda i: (0, 0)),
            pl.BlockSpec((1, NB), lambda i: (0, i)),
        ],
        out_specs=[
            pl.BlockSpec((NB, D), lambda i: (i, 0)),
            pl.BlockSpec((G, D), lambda i: (0, 0)),
            pl.BlockSpec((G, D), lambda i: (0, 0)),
        ],
        out_shape=[
            jax.ShapeDtypeStruct((NP_, D), jnp.float32),
            jax.ShapeDtypeStruct((G, D), jnp.float32),
            jax.ShapeDtypeStruct((G, D), jnp.float32),
        ],
    )(a0, a1, hp, dinv, bias, batch2)


def _gn_apply(y, batch_blk, valid, s1, s2, cnt, w, b, ms):
    cntc = jnp.maximum(jnp.transpose(cnt, (1, 0)), 1.0)
    mean = s1 / cntc
    m2 = s2 / cntc
    var = m2 - (2.0 * ms - ms * ms) * mean * mean
    oh = _onehot(batch_blk, valid)
    mean_r = jnp.dot(oh, mean, preferred_element_type=jnp.float32, precision=lax.Precision.HIGHEST)
    var_r = jnp.dot(oh, var, preferred_element_type=jnp.float32, precision=lax.Precision.HIGHEST)
    out = y - mean_r * ms
    return w * out * lax.rsqrt(var_r + EPS) + b


def _tc_gn1_kernel(y_ref, s1_ref, s2_ref, cnt_ref, w_ref, b_ref, ms_ref,
                   batch_ref, dinv_ref, w2_ref, hp2_ref):
    i = pl.program_id(0)
    valid = _rows_valid(i)
    z = _gn_apply(y_ref[...], batch_ref[...], valid, s1_ref[...], s2_ref[...],
                  cnt_ref[...], w_ref[...], b_ref[...], ms_ref[...])
    z = jnp.maximum(z, 0.0)
    dv = jnp.transpose(dinv_ref[...], (1, 0))
    hp2_ref[...] = dv * jnp.dot(z, w2_ref[...],
                                preferred_element_type=jnp.float32)


def _tc_gn1(y, s1, s2, cnt, w, b, ms, batch2, dinv, w2):
    return pl.pallas_call(
        _tc_gn1_kernel,
        grid=(NBLK,),
        in_specs=[
            pl.BlockSpec((NB, D), lambda i: (i, 0)),
            pl.BlockSpec((G, D), lambda i: (0, 0)),
            pl.BlockSpec((G, D), lambda i: (0, 0)),
            pl.BlockSpec((1, G), lambda i: (0, 0)),
            pl.BlockSpec((1, D), lambda i: (0, 0)),
            pl.BlockSpec((1, D), lambda i: (0, 0)),
            pl.BlockSpec((1, D), lambda i: (0, 0)),
            pl.BlockSpec((1, NB), lambda i: (0, i)),
            pl.BlockSpec((1, NB), lambda i: (0, i)),
            pl.BlockSpec((D, D), lambda i: (0, 0)),
        ],
        out_specs=pl.BlockSpec((NB, D), lambda i: (i, 0)),
        out_shape=jax.ShapeDtypeStruct((NP_, D), jnp.float32),
    )(y, s1, s2, cnt, w, b, ms, batch2, dinv, w2)


def _tc_gn2_kernel(y_ref, s1_ref, s2_ref, cnt_ref, w_ref, b_ref, ms_ref,
                   batch_ref, gw1_ref, gb1_ref,
                   h_ref, t_ref, st1_ref, st2_ref):
    i = pl.program_id(0)
    valid = _rows_valid(i)
    z = _gn_apply(y_ref[...], batch_ref[...], valid, s1_ref[...], s2_ref[...],
                  cnt_ref[...], w_ref[...], b_ref[...], ms_ref[...])
    h = jnp.maximum(z, 0.0)
    h_ref[...] = h
    t = jnp.dot(h, gw1_ref[...], preferred_element_type=jnp.float32) + gb1_ref[...]
    t_ref[...] = t
    tm = jnp.where(valid, t, 0.0)
    @pl.when(i == 0)
    def _():
        st1_ref[...] = jnp.zeros_like(st1_ref)
        st2_ref[...] = jnp.zeros_like(st2_ref)
    st1_ref[...] += jnp.sum(tm, axis=0, keepdims=True)
    st2_ref[...] += jnp.sum(tm * tm, axis=0, keepdims=True)


def _tc_gn2(y, s1, s2, cnt, w, b, ms, batch2, gw1, gb1):
    return pl.pallas_call(
        _tc_gn2_kernel,
        grid=(NBLK,),
        in_specs=[
            pl.BlockSpec((NB, D), lambda i: (i, 0)),
            pl.BlockSpec((G, D), lambda i: (0, 0)),
            pl.BlockSpec((G, D), lambda i: (0, 0)),
            pl.BlockSpec((1, G), lambda i: (0, 0)),
            pl.BlockSpec((1, D), lambda i: (0, 0)),
            pl.BlockSpec((1, D), lambda i: (0, 0)),
            pl.BlockSpec((1, D), lambda i: (0, 0)),
            pl.BlockSpec((1, NB), lambda i: (0, i)),
            pl.BlockSpec((D, 2 * D), lambda i: (0, 0)),
            pl.BlockSpec((1, 2 * D), lambda i: (0, 0)),
        ],
        out_specs=[
            pl.BlockSpec((NB, D), lambda i: (i, 0)),
            pl.BlockSpec((NB, 2 * D), lambda i: (i, 0)),
            pl.BlockSpec((1, 2 * D), lambda i: (0, 0)),
            pl.BlockSpec((1, 2 * D), lambda i: (0, 0)),
        ],
        out_shape=[
            jax.ShapeDtypeStruct((NP_, D), jnp.float32),
            jax.ShapeDtypeStruct((NP_, 2 * D), jnp.float32),
            jax.ShapeDtypeStruct((1, 2 * D), jnp.float32),
            jax.ShapeDtypeStruct((1, 2 * D), jnp.float32),
        ],
    )(y, s1, s2, cnt, w, b, ms, batch2, gw1, gb1)


def _tc_gate_kernel(t_ref, st1_ref, st2_ref, g_ref, b_ref, gw2_ref, gb2_ref,
                    batch_ref, gate_ref, gmax_ref):
    i = pl.program_id(0)
    valid = _rows_valid(i)
    m = st1_ref[...] / float(N)
    v = st2_ref[...] / float(N) - m * m
    tb = g_ref[...] * (t_ref[...] - m) * lax.rsqrt(v + EPS) + b_ref[...]
    tb = jnp.maximum(tb, 0.0)
    gate = jnp.dot(tb, gw2_ref[...], preferred_element_type=jnp.float32) \
        + gb2_ref[...]
    gate_ref[...] = jnp.transpose(gate, (1, 0))
    oh = _onehot(batch_ref[...], valid)
    gm = jnp.max(jnp.where(oh > 0, gate, -3e38), axis=0, keepdims=True)
    @pl.when(i == 0)
    def _():
        gmax_ref[...] = jnp.full_like(gmax_ref, -3e38)
    gmax_ref[...] = jnp.maximum(gmax_ref[...], gm)


def _tc_gate(t, st1, st2, g, b, gw2, gb2, batch2):
    return pl.pallas_call(
        _tc_gate_kernel,
        grid=(NBLK,),
        in_specs=[
            pl.BlockSpec((NB, 2 * D), lambda i: (i, 0)),
            pl.BlockSpec((1, 2 * D), lambda i: (0, 0)),
            pl.BlockSpec((1, 2 * D), lambda i: (0, 0)),
            pl.BlockSpec((1, 2 * D), lambda i: (0, 0)),
            pl.BlockSpec((1, 2 * D), lambda i: (0, 0)),
            pl.BlockSpec((2 * D, 1), lambda i: (0, 0)),
            pl.BlockSpec((1, 1), lambda i: (0, 0)),
            pl.BlockSpec((1, NB), lambda i: (0, i)),
        ],
        out_specs=[
            pl.BlockSpec((1, NB), lambda i: (0, i)),
            pl.BlockSpec((1, G), lambda i: (0, 0)),
        ],
        out_shape=[
            jax.ShapeDtypeStruct((1, NP_), jnp.float32),
            jax.ShapeDtypeStruct((1, G), jnp.float32),
        ],
    )(t, st1, st2, g, b, gw2, gb2, batch2)


def _tc_pool_kernel(gate_ref, gmax_ref, h_ref, batch_ref, num_ref, den_ref):
    i = pl.program_id(0)
    valid = _rows_valid(i)
    oh = _onehot(batch_ref[...], valid)
    gmax_r = jnp.dot(oh, jnp.transpose(gmax_ref[...], (1, 0)),
                     preferred_element_type=jnp.float32, precision=lax.Precision.HIGHEST)
    gate = jnp.transpose(gate_ref[...], (1, 0))
    e = jnp.where(valid, jnp.exp(gate - gmax_r), 0.0)
    ohT = jnp.transpose(oh, (1, 0))
    @pl.when(i == 0)
    def _():
        num_ref[...] = jnp.zeros_like(num_ref)
        den_ref[...] = jnp.zeros_like(den_ref)
    num_ref[...] += jnp.dot(ohT, e * h_ref[...],
                            preferred_element_type=jnp.float32, precision=lax.Precision.HIGHEST)
    den_ref[...] += jnp.sum(oh * e, axis=0, keepdims=True)


def _tc_pool(gate, gmax, h, batch2):
    return pl.pallas_call(
        _tc_pool_kernel,
        grid=(NBLK,),
        in_specs=[
            pl.BlockSpec((1, NB), lambda i: (0, i)),
            pl.BlockSpec((1, G), lambda i: (0, 0)),
            pl.BlockSpec((NB, D), lambda i: (i, 0)),
            pl.BlockSpec((1, NB), lambda i: (0, i)),
        ],
        out_specs=[
            pl.BlockSpec((G, D), lambda i: (0, 0)),
            pl.BlockSpec((1, G), lambda i: (0, 0)),
        ],
        out_shape=[
            jax.ShapeDtypeStruct((G, D), jnp.float32),
            jax.ShapeDtypeStruct((1, G), jnp.float32),
        ],
    )(gate, gmax, h, batch2)


def _bn_rows(x, g, b):
    m = jnp.mean(x, axis=0, keepdims=True)
    v = jnp.mean(x * x, axis=0, keepdims=True) - m * m
    return g * (x - m) * lax.rsqrt(v + EPS) + b


def _tc_head_kernel(num_ref, den_ref, lw1_ref, lb1_ref, g1_ref, bb1_ref,
                    lw2_ref, lb2_ref, g2_ref, bb2_ref, lw3_ref, lb3_ref,
                    out_ref):
    den = jnp.maximum(jnp.transpose(den_ref[...], (1, 0)), 1e-16)
    pooled = num_ref[...] / den
    z = jnp.dot(pooled, lw1_ref[...], preferred_element_type=jnp.float32) \
        + lb1_ref[...]
    z = jnp.maximum(_bn_rows(z, g1_ref[...], bb1_ref[...]), 0.0)
    z = jnp.dot(z, lw2_ref[...], preferred_element_type=jnp.float32) \
        + lb2_ref[...]
    z = jnp.maximum(_bn_rows(z, g2_ref[...], bb2_ref[...]), 0.0)
    out_ref[...] = jnp.dot(z, lw3_ref[...],
                           preferred_element_type=jnp.float32) + lb3_ref[...]


def _tc_head(num, den, lw1, lb1, g1, bb1, lw2, lb2, g2, bb2, lw3, lb3):
    specs = [
        pl.BlockSpec((G, D), lambda: (0, 0)),
        pl.BlockSpec((1, G), lambda: (0, 0)),
        pl.BlockSpec((D, LIN), lambda: (0, 0)),
        pl.BlockSpec((1, LIN), lambda: (0, 0)),
        pl.BlockSpec((1, LIN), lambda: (0, 0)),
        pl.BlockSpec((1, LIN), lambda: (0, 0)),
        pl.BlockSpec((LIN, LIN), lambda: (0, 0)),
        pl.BlockSpec((1, LIN), lambda: (0, 0)),
        pl.BlockSpec((1, LIN), lambda: (0, 0)),
        pl.BlockSpec((1, LIN), lambda: (0, 0)),
        pl.BlockSpec((LIN, 1), lambda: (0, 0)),
        pl.BlockSpec((1, 1), lambda: (0, 0)),
    ]
    return pl.pallas_call(
        _tc_head_kernel,
        in_specs=specs,
        out_specs=pl.BlockSpec((G, 1), lambda: (0, 0)),
        out_shape=jax.ShapeDtypeStruct((G, 1), jnp.float32),
    )(num, den, lw1, lb1, g1, bb1, lw2, lb2, g2, bb2, lw3, lb3)


def kernel(x, edge_attr, edge_index, batch,
           W1, b1, gn1_w, gn1_b, gn1_ms,
           W2, b2, gn2_w, gn2_b, gn2_ms,
           gW1, gb1, gbn_g, gbn_b, gW2, gb2,
           lW1, lb1, lbn1_g, lbn1_b, lW2, lb2, lbn2_g, lbn2_b, lW3, lb3):
    f32 = jnp.float32
    row = edge_index[0].astype(jnp.int32)
    col = edge_index[1].astype(jnp.int32)
    ew = edge_attr.astype(f32)

    pad = EP - E
    rowf = jnp.concatenate([row, jnp.zeros((pad,), jnp.int32)])
    colf = jnp.concatenate([col, jnp.zeros((pad,), jnp.int32)])
    ewf = jnp.concatenate([ew, jnp.zeros((pad,), f32)])
    ewbits = lax.bitcast_convert_type(ewf, jnp.int32)
    rce = jnp.concatenate([rowf.reshape(NW, KH, 2, C),
                           colf.reshape(NW, KH, 2, C),
                           ewbits.reshape(NW, KH, 2, C)], axis=2)

    xp = jnp.pad(x.astype(f32), ((0, NP_ - N), (0, 0)))
    batchp = jnp.pad(batch.astype(jnp.int32), (0, NP_ - N),
                     constant_values=G).reshape(1, NP_)

    r2 = lambda a: a.astype(f32).reshape(1, -1)

    sc_deg = _make_sc_deg()

    sc_agg = _make_sc_agg()

    dacc = sc_deg(rce)
    hp1, dinv, cnt = _tc_prep(xp, W1.astype(f32), dacc[0], dacc[1], batchp)

    a1 = sc_agg(rce, hp1)
    y1, s1, s2 = _tc_stats(a1[0], a1[1], hp1, dinv, r2(b1), batchp)
    hp2 = _tc_gn1(y1, s1, s2, cnt, r2(gn1_w), r2(gn1_b), r2(gn1_ms),
                  batchp, dinv, W2.astype(f32))

    a2 = sc_agg(rce, hp2)
    y2, s1b, s2b = _tc_stats(a2[0], a2[1], hp2, dinv, r2(b2), batchp)
    h, t, st1, st2 = _tc_gn2(y2, s1b, s2b, cnt, r2(gn2_w), r2(gn2_b),
                             r2(gn2_ms), batchp, gW1.astype(f32), r2(gb1))

    gate, gmax = _tc_gate(t, st1, st2, r2(gbn_g), r2(gbn_b),
                          gW2.astype(f32), gb2.reshape(1, 1).astype(f32),
                          batchp)
    num, den = _tc_pool(gate, gmax, h, batchp)

    out = _tc_head(num, den, lW1.astype(f32), r2(lb1), r2(lbn1_g), r2(lbn1_b),
                   lW2.astype(f32), r2(lb2), r2(lbn2_g), r2(lbn2_b),
                   lW3.astype(f32), lb3.reshape(1, 1).astype(f32))
    return out


# speedup vs baseline: 6.6792x; 1.1067x over previous
import functools
import jax
import jax.numpy as jnp
from jax import lax
from jax.experimental import pallas as pl
from jax.experimental.pallas import tpu as pltpu, tpu_sc as plsc

N = 10000
E = 320000
D = 128
G = 64
LIN = 256
EPS = 1e-5

NP_ = 10240
NB = 256
NBLK = NP_ // NB
NC = 2
NS = 16
NW = NC * NS
C = 128
EW_PER = 10240
KC = EW_PER // C
CA = 64
KA = EW_PER // CA
EP = NW * EW_PER
ROWS_PER_TILE = NP_ // NS
KH = KC // 2
NAGG = 10112
AGG_ROWS = NAGG // NS


def _sc_deg_body(rce_hbm, out_hbm, stg, ewb, acc_sh, sem):
    cid = lax.axis_index("c")
    sid = lax.axis_index("s")
    wid = cid * NS + sid

    def zrow(r, _):
        for g in range(D // 16):
            ewb[r, pl.ds(g * 16, 16)] = jnp.zeros((16,), jnp.float32)
        return 0
    lax.fori_loop(0, C, zrow, 0)
    base = sid * ROWS_PER_TILE
    for z in range(ROWS_PER_TILE // C):
        pltpu.sync_copy(ewb, acc_sh.at[pl.ds(base + z * C, C)])
    plsc.subcore_barrier()

    lane = lax.iota(jnp.int32, 16)

    def chunk(jh, _):
        pltpu.sync_copy(rce_hbm.at[wid, jh], stg)
        for h in range(2):
            def fill(i2, _):
                ev = plsc.bitcast(stg[4 + h, pl.ds(i2 * 16, 16)], jnp.float32)
                for k in range(16):
                    ewb[i2 * 16 + k, pl.ds(0, 16)] = jnp.where(
                        lane == k, ev[k], 0.0)
                return 0
            lax.fori_loop(0, C // 16, fill, 0)
            pltpu.sync_copy(ewb, acc_sh.at[stg.at[2 + h]], add=True)
        return 0
    lax.fori_loop(0, KH, chunk, 0)
    plsc.subcore_barrier()

    pltpu.sync_copy(acc_sh.at[pl.ds(base, ROWS_PER_TILE)],
                    out_hbm.at[cid, pl.ds(base, ROWS_PER_TILE)])


def _sc_agg_body(rce_hbm, hp_hbm, out_hbm, stg, buf, acc_sh, sem):
    cid = lax.axis_index("c")
    sid = lax.axis_index("s")
    wid = cid * NS + sid

    def zrow(r, _):
        for g in range(D // 16):
            buf[r, pl.ds(g * 16, 16)] = jnp.zeros((16,), jnp.float32)
        return 0
    lax.fori_loop(0, C, zrow, 0)
    base = sid * ROWS_PER_TILE
    for z in range(ROWS_PER_TILE // C):
        pltpu.sync_copy(buf, acc_sh.at[pl.ds(base + z * C, C)])
    plsc.subcore_barrier()

    def chunk(jh, _):
        pltpu.sync_copy(rce_hbm.at[wid, jh], stg)
        for h in range(2):
            pltpu.async_copy(hp_hbm.at[stg.at[h]], buf, sem).wait()

            def scale(i2, _):
                ev = plsc.bitcast(stg[4 + h, pl.ds(i2 * 16, 16)], jnp.float32)
                for k in range(16):
                    sc = ev[k]
                    r = i2 * 16 + k
                    for g in range(D // 16):
                        buf[r, pl.ds(g * 16, 16)] = (
                            buf[r, pl.ds(g * 16, 16)] * sc)
                return 0
            lax.fori_loop(0, C // 16, scale, 0)
            pltpu.sync_copy(buf, acc_sh.at[stg.at[2 + h]], add=True)
        return 0
    lax.fori_loop(0, KH, chunk, 0)
    plsc.subcore_barrier()

    pltpu.sync_copy(acc_sh.at[pl.ds(base, ROWS_PER_TILE)],
                    out_hbm.at[cid, pl.ds(base, ROWS_PER_TILE)])


def _make_sc_deg():
    mesh = plsc.VectorSubcoreMesh(core_axis_name="c", subcore_axis_name="s",
                                  num_cores=NC, num_subcores=NS)
    return pl.kernel(
        _sc_deg_body,
        out_type=jax.ShapeDtypeStruct((NC, NP_, D), jnp.float32),
        mesh=mesh,
        compiler_params=pltpu.CompilerParams(needs_layout_passes=False),
        scratch_types=[
            pltpu.VMEM((6, C), jnp.int32),
            pltpu.VMEM((C, D), jnp.float32),
            pltpu.VMEM_SHARED((NP_, D), jnp.float32),
            pltpu.SemaphoreType.DMA,
        ],
    )


def _make_sc_agg():
    mesh = plsc.VectorSubcoreMesh(core_axis_name="c", subcore_axis_name="s",
                                  num_cores=NC, num_subcores=NS)
    return pl.kernel(
        _sc_agg_body,
        out_type=jax.ShapeDtypeStruct((NC, NP_, D), jnp.float32),
        mesh=mesh,
        compiler_params=pltpu.CompilerParams(needs_layout_passes=False),
        scratch_types=[
            pltpu.VMEM((6, C), jnp.int32),
            pltpu.VMEM((C, D), jnp.float32),
            pltpu.VMEM_SHARED((NP_, D), jnp.float32),
            pltpu.SemaphoreType.DMA,
        ],
    )


def _rows_valid(i):
    ridx = i * NB + lax.broadcasted_iota(jnp.int32, (NB, 1), 0)
    return ridx < N


def _onehot(batch_blk, valid):
    b = jnp.transpose(batch_blk, (1, 0))
    gi = lax.broadcasted_iota(jnp.int32, (1, G), 1)
    return jnp.where((b == gi) & valid, 1.0, 0.0).astype(jnp.float32)


def _tc_prep_kernel(x_ref, w_ref, d0_ref, d1_ref, batch_ref,
                    hp_ref, dinv_ref, cnt_ref):
    i = pl.program_id(0)
    deg = 1.0 + jnp.sum(d0_ref[...] + d1_ref[...], axis=1, keepdims=True)
    dinv = jnp.where(deg > 0, lax.rsqrt(jnp.maximum(deg, 1e-12)), 0.0)
    hp_ref[...] = dinv * jnp.dot(x_ref[...], w_ref[...],
                                 preferred_element_type=jnp.float32)
    dinv_ref[...] = jnp.transpose(dinv, (1, 0))
    valid = _rows_valid(i)
    oh = _onehot(batch_ref[...], valid)
    c = jnp.sum(oh, axis=0, keepdims=True)
    @pl.when(i == 0)
    def _():
        cnt_ref[...] = jnp.zeros_like(cnt_ref)
    cnt_ref[...] += c


def _tc_prep(x, w, d0, d1, batch2):
    return pl.pallas_call(
        _tc_prep_kernel,
        grid=(NBLK,),
        in_specs=[
            pl.BlockSpec((NB, D), lambda i: (i, 0)),
            pl.BlockSpec((D, D), lambda i: (0, 0)),
            pl.BlockSpec((NB, D), lambda i: (i, 0)),
            pl.BlockSpec((NB, D), lambda i: (i, 0)),
            pl.BlockSpec((1, NB), lambda i: (0, i)),
        ],
        out_specs=[
            pl.BlockSpec((NB, D), lambda i: (i, 0)),
            pl.BlockSpec((1, NB), lambda i: (0, i)),
            pl.BlockSpec((1, G), lambda i: (0, 0)),
        ],
        out_shape=[
            jax.ShapeDtypeStruct((NP_, D), jnp.float32),
            jax.ShapeDtypeStruct((1, NP_), jnp.float32),
            jax.ShapeDtypeStruct((1, G), jnp.float32),
        ],
    )(x, w, d0, d1, batch2)


def _tc_stats_kernel(a0_ref, a1_ref, hp_ref, dinv_ref, b_ref, batch_ref,
                     y_ref, s1_ref, s2_ref):
    i = pl.program_id(0)
    dv = jnp.transpose(dinv_ref[...], (1, 0))
    valid = _rows_valid(i)
    y = dv * (a0_ref[...] + a1_ref[...] + hp_ref[...]) + b_ref[...]
    y = jnp.where(valid, y, 0.0)
    y_ref[...] = y
    oh = _onehot(batch_ref[...], valid)
    ohT = jnp.transpose(oh, (1, 0))
    @pl.when(i == 0)
    def _():
        s1_ref[...] = jnp.zeros_like(s1_ref)
        s2_ref[...] = jnp.zeros_like(s2_ref)
    s1_ref[...] += jnp.dot(ohT, y, preferred_element_type=jnp.float32, precision=lax.Precision.HIGHEST)
    s2_ref[...] += jnp.dot(ohT, y * y, preferred_element_type=jnp.float32, precision=lax.Precision.HIGHEST)


def _tc_stats(a0, a1, hp, dinv, bias, batch2):
    return pl.pallas_call(
        _tc_stats_kernel,
        grid=(NBLK,),
        in_specs=[
            pl.BlockSpec((NB, D), lambda i: (i, 0)),
            pl.BlockSpec((NB, D), lambda i: (i, 0)),
            pl.BlockSpec((NB, D), lambda i: (i, 0)),
            pl.BlockSpec((1, NB), lambda i: (0, i)),
            pl.BlockSpec((1, D), lambda i: (0, 0)),
            pl.BlockSpec((1, NB), lambda i: (0, i)),
        ],
        out_specs=[
            pl.BlockSpec((NB, D), lambda i: (i, 0)),
            pl.BlockSpec((G, D), lambda i: (0, 0)),
            pl.BlockSpec((G, D), lambda i: (0, 0)),
        ],
        out_shape=[
            jax.ShapeDtypeStruct((NP_, D), jnp.float32),
            jax.ShapeDtypeStruct((G, D), jnp.float32),
            jax.ShapeDtypeStruct((G, D), jnp.float32),
        ],
    )(a0, a1, hp, dinv, bias, batch2)


def _gn_apply(y, batch_blk, valid, s1, s2, cnt, w, b, ms):
    cntc = jnp.maximum(jnp.transpose(cnt, (1, 0)), 1.0)
    mean = s1 / cntc
    m2 = s2 / cntc
    var = m2 - (2.0 * ms - ms * ms) * mean * mean
    oh = _onehot(batch_blk, valid)
    mean_r = jnp.dot(oh, mean, preferred_element_type=jnp.float32, precision=lax.Precision.HIGHEST)
    var_r = jnp.dot(oh, var, preferred_element_type=jnp.float32, precision=lax.Precision.HIGHEST)
    out = y - mean_r * ms
    return w * out * lax.rsqrt(var_r + EPS) + b


def _tc_gn1_kernel(y_ref, s1_ref, s2_ref, cnt_ref, w_ref, b_ref, ms_ref,
                   batch_ref, dinv_ref, w2_ref, hp2_ref):
    i = pl.program_id(0)
    valid = _rows_valid(i)
    z = _gn_apply(y_ref[...], batch_ref[...], valid, s1_ref[...], s2_ref[...],
                  cnt_ref[...], w_ref[...], b_ref[...], ms_ref[...])
    z = jnp.maximum(z, 0.0)
    dv = jnp.transpose(dinv_ref[...], (1, 0))
    hp2_ref[...] = dv * jnp.dot(z, w2_ref[...],
                                preferred_element_type=jnp.float32)


def _tc_gn1(y, s1, s2, cnt, w, b, ms, batch2, dinv, w2):
    return pl.pallas_call(
        _tc_gn1_kernel,
        grid=(NBLK,),
        in_specs=[
            pl.BlockSpec((NB, D), lambda i: (i, 0)),
            pl.BlockSpec((G, D), lambda i: (0, 0)),
            pl.BlockSpec((G, D), lambda i: (0, 0)),
            pl.BlockSpec((1, G), lambda i: (0, 0)),
            pl.BlockSpec((1, D), lambda i: (0, 0)),
            pl.BlockSpec((1, D), lambda i: (0, 0)),
            pl.BlockSpec((1, D), lambda i: (0, 0)),
            pl.BlockSpec((1, NB), lambda i: (0, i)),
            pl.BlockSpec((1, NB), lambda i: (0, i)),
            pl.BlockSpec((D, D), lambda i: (0, 0)),
        ],
        out_specs=pl.BlockSpec((NB, D), lambda i: (i, 0)),
        out_shape=jax.ShapeDtypeStruct((NP_, D), jnp.float32),
    )(y, s1, s2, cnt, w, b, ms, batch2, dinv, w2)


def _tc_gn2_kernel(y_ref, s1_ref, s2_ref, cnt_ref, w_ref, b_ref, ms_ref,
                   batch_ref, gw1_ref, gb1_ref,
                   h_ref, t_ref, st1_ref, st2_ref):
    i = pl.program_id(0)
    valid = _rows_valid(i)
    z = _gn_apply(y_ref[...], batch_ref[...], valid, s1_ref[...], s2_ref[...],
                  cnt_ref[...], w_ref[...], b_ref[...], ms_ref[...])
    h = jnp.maximum(z, 0.0)
    h_ref[...] = h
    t = jnp.dot(h, gw1_ref[...], preferred_element_type=jnp.float32) + gb1_ref[...]
    t_ref[...] = t
    tm = jnp.where(valid, t, 0.0)
    @pl.when(i == 0)
    def _():
        st1_ref[...] = jnp.zeros_like(st1_ref)
        st2_ref[...] = jnp.zeros_like(st2_ref)
    st1_ref[...] += jnp.sum(tm, axis=0, keepdims=True)
    st2_ref[...] += jnp.sum(tm * tm, axis=0, keepdims=True)


def _tc_gn2(y, s1, s2, cnt, w, b, ms, batch2, gw1, gb1):
    return pl.pallas_call(
        _tc_gn2_kernel,
        grid=(NBLK,),
        in_specs=[
            pl.BlockSpec((NB, D), lambda i: (i, 0)),
            pl.BlockSpec((G, D), lambda i: (0, 0)),
            pl.BlockSpec((G, D), lambda i: (0, 0)),
            pl.BlockSpec((1, G), lambda i: (0, 0)),
            pl.BlockSpec((1, D), lambda i: (0, 0)),
            pl.BlockSpec((1, D), lambda i: (0, 0)),
            pl.BlockSpec((1, D), lambda i: (0, 0)),
            pl.BlockSpec((1, NB), lambda i: (0, i)),
            pl.BlockSpec((D, 2 * D), lambda i: (0, 0)),
            pl.BlockSpec((1, 2 * D), lambda i: (0, 0)),
        ],
        out_specs=[
            pl.BlockSpec((NB, D), lambda i: (i, 0)),
            pl.BlockSpec((NB, 2 * D), lambda i: (i, 0)),
            pl.BlockSpec((1, 2 * D), lambda i: (0, 0)),
            pl.BlockSpec((1, 2 * D), lambda i: (0, 0)),
        ],
        out_shape=[
            jax.ShapeDtypeStruct((NP_, D), jnp.float32),
            jax.ShapeDtypeStruct((NP_, 2 * D), jnp.float32),
            jax.ShapeDtypeStruct((1, 2 * D), jnp.float32),
            jax.ShapeDtypeStruct((1, 2 * D), jnp.float32),
        ],
    )(y, s1, s2, cnt, w, b, ms, batch2, gw1, gb1)


def _tc_gate_kernel(t_ref, st1_ref, st2_ref, g_ref, b_ref, gw2_ref, gb2_ref,
                    batch_ref, gate_ref, gmax_ref):
    i = pl.program_id(0)
    valid = _rows_valid(i)
    m = st1_ref[...] / float(N)
    v = st2_ref[...] / float(N) - m * m
    tb = g_ref[...] * (t_ref[...] - m) * lax.rsqrt(v + EPS) + b_ref[...]
    tb = jnp.maximum(tb, 0.0)
    gate = jnp.dot(tb, gw2_ref[...], preferred_element_type=jnp.float32) \
        + gb2_ref[...]
    gate_ref[...] = jnp.transpose(gate, (1, 0))
    oh = _onehot(batch_ref[...], valid)
    gm = jnp.max(jnp.where(oh > 0, gate, -3e38), axis=0, keepdims=True)
    @pl.when(i == 0)
    def _():
        gmax_ref[...] = jnp.full_like(gmax_ref, -3e38)
    gmax_ref[...] = jnp.maximum(gmax_ref[...], gm)


def _tc_gate(t, st1, st2, g, b, gw2, gb2, batch2):
    return pl.pallas_call(
        _tc_gate_kernel,
        grid=(NBLK,),
        in_specs=[
            pl.BlockSpec((NB, 2 * D), lambda i: (i, 0)),
            pl.BlockSpec((1, 2 * D), lambda i: (0, 0)),
            pl.BlockSpec((1, 2 * D), lambda i: (0, 0)),
            pl.BlockSpec((1, 2 * D), lambda i: (0, 0)),
            pl.BlockSpec((1, 2 * D), lambda i: (0, 0)),
            pl.BlockSpec((2 * D, 1), lambda i: (0, 0)),
            pl.BlockSpec((1, 1), lambda i: (0, 0)),
            pl.BlockSpec((1, NB), lambda i: (0, i)),
        ],
        out_specs=[
            pl.BlockSpec((1, NB), lambda i: (0, i)),
            pl.BlockSpec((1, G), lambda i: (0, 0)),
        ],
        out_shape=[
            jax.ShapeDtypeStruct((1, NP_), jnp.float32),
            jax.ShapeDtypeStruct((1, G), jnp.float32),
        ],
    )(t, st1, st2, g, b, gw2, gb2, batch2)


def _tc_pool_kernel(gate_ref, gmax_ref, h_ref, batch_ref, num_ref, den_ref):
    i = pl.program_id(0)
    valid = _rows_valid(i)
    oh = _onehot(batch_ref[...], valid)
    gmax_r = jnp.dot(oh, jnp.transpose(gmax_ref[...], (1, 0)),
                     preferred_element_type=jnp.float32, precision=lax.Precision.HIGHEST)
    gate = jnp.transpose(gate_ref[...], (1, 0))
    e = jnp.where(valid, jnp.exp(gate - gmax_r), 0.0)
    ohT = jnp.transpose(oh, (1, 0))
    @pl.when(i == 0)
    def _():
        num_ref[...] = jnp.zeros_like(num_ref)
        den_ref[...] = jnp.zeros_like(den_ref)
    num_ref[...] += jnp.dot(ohT, e * h_ref[...],
                            preferred_element_type=jnp.float32, precision=lax.Precision.HIGHEST)
    den_ref[...] += jnp.sum(oh * e, axis=0, keepdims=True)


def _tc_pool(gate, gmax, h, batch2):
    return pl.pallas_call(
        _tc_pool_kernel,
        grid=(NBLK,),
        in_specs=[
            pl.BlockSpec((1, NB), lambda i: (0, i)),
            pl.BlockSpec((1, G), lambda i: (0, 0)),
            pl.BlockSpec((NB, D), lambda i: (i, 0)),
            pl.BlockSpec((1, NB), lambda i: (0, i)),
        ],
        out_specs=[
            pl.BlockSpec((G, D), lambda i: (0, 0)),
            pl.BlockSpec((1, G), lambda i: (0, 0)),
        ],
        out_shape=[
            jax.ShapeDtypeStruct((G, D), jnp.float32),
            jax.ShapeDtypeStruct((1, G), jnp.float32),
        ],
    )(gate, gmax, h, batch2)


def _bn_rows(x, g, b):
    m = jnp.mean(x, axis=0, keepdims=True)
    v = jnp.mean(x * x, axis=0, keepdims=True) - m * m
    return g * (x - m) * lax.rsqrt(v + EPS) + b


def _tc_head_kernel(num_ref, den_ref, lw1_ref, lb1_ref, g1_ref, bb1_ref,
                    lw2_ref, lb2_ref, g2_ref, bb2_ref, lw3_ref, lb3_ref,
                    out_ref):
    den = jnp.maximum(jnp.transpose(den_ref[...], (1, 0)), 1e-16)
    pooled = num_ref[...] / den
    z = jnp.dot(pooled, lw1_ref[...], preferred_element_type=jnp.float32) \
        + lb1_ref[...]
    z = jnp.maximum(_bn_rows(z, g1_ref[...], bb1_ref[...]), 0.0)
    z = jnp.dot(z, lw2_ref[...], preferred_element_type=jnp.float32) \
        + lb2_ref[...]
    z = jnp.maximum(_bn_rows(z, g2_ref[...], bb2_ref[...]), 0.0)
    out_ref[...] = jnp.dot(z, lw3_ref[...],
                           preferred_element_type=jnp.float32) + lb3_ref[...]


def _tc_head(num, den, lw1, lb1, g1, bb1, lw2, lb2, g2, bb2, lw3, lb3):
    specs = [
        pl.BlockSpec((G, D), lambda: (0, 0)),
        pl.BlockSpec((1, G), lambda: (0, 0)),
        pl.BlockSpec((D, LIN), lambda: (0, 0)),
        pl.BlockSpec((1, LIN), lambda: (0, 0)),
        pl.BlockSpec((1, LIN), lambda: (0, 0)),
        pl.BlockSpec((1, LIN), lambda: (0, 0)),
        pl.BlockSpec((LIN, LIN), lambda: (0, 0)),
        pl.BlockSpec((1, LIN), lambda: (0, 0)),
        pl.BlockSpec((1, LIN), lambda: (0, 0)),
        pl.BlockSpec((1, LIN), lambda: (0, 0)),
        pl.BlockSpec((LIN, 1), lambda: (0, 0)),
        pl.BlockSpec((1, 1), lambda: (0, 0)),
    ]
    return pl.pallas_call(
        _tc_head_kernel,
        in_specs=specs,
        out_specs=pl.BlockSpec((G, 1), lambda: (0, 0)),
        out_shape=jax.ShapeDtypeStruct((G, 1), jnp.float32),
    )(num, den, lw1, lb1, g1, bb1, lw2, lb2, g2, bb2, lw3, lb3)


def kernel(x, edge_attr, edge_index, batch,
           W1, b1, gn1_w, gn1_b, gn1_ms,
           W2, b2, gn2_w, gn2_b, gn2_ms,
           gW1, gb1, gbn_g, gbn_b, gW2, gb2,
           lW1, lb1, lbn1_g, lbn1_b, lW2, lb2, lbn2_g, lbn2_b, lW3, lb3):
    f32 = jnp.float32
    row = edge_index[0].astype(jnp.int32)
    col = edge_index[1].astype(jnp.int32)
    ew = edge_attr.astype(f32)

    pad = EP - E
    rowf = jnp.concatenate([row, jnp.zeros((pad,), jnp.int32)])
    colf = jnp.concatenate([col, jnp.zeros((pad,), jnp.int32)])
    ewf = jnp.concatenate([ew, jnp.zeros((pad,), f32)])
    ewbits = lax.bitcast_convert_type(ewf, jnp.int32)
    rce = jnp.concatenate([rowf.reshape(NW, KH, 2, C),
                           colf.reshape(NW, KH, 2, C),
                           ewbits.reshape(NW, KH, 2, C)], axis=2)

    xp = jnp.pad(x.astype(f32), ((0, NP_ - N), (0, 0)))
    batchp = jnp.pad(batch.astype(jnp.int32), (0, NP_ - N),
                     constant_values=G).reshape(1, NP_)

    r2 = lambda a: a.astype(f32).reshape(1, -1)

    sc_deg = _make_sc_deg()

    sc_agg = _make_sc_agg()

    dacc = sc_deg(rce)
    hp1, dinv, cnt = _tc_prep(xp, W1.astype(f32), dacc[0], dacc[1], batchp)

    a1 = sc_agg(rce, hp1)
    y1, s1, s2 = _tc_stats(a1[0], a1[1], hp1, dinv, r2(b1), batchp)
    hp2 = _tc_gn1(y1, s1, s2, cnt, r2(gn1_w), r2(gn1_b), r2(gn1_ms),
                  batchp, dinv, W2.astype(f32))

    a2 = sc_agg(rce, hp2)
    y2, s1b, s2b = _tc_stats(a2[0], a2[1], hp2, dinv, r2(b2), batchp)
    h, t, st1, st2 = _tc_gn2(y2, s1b, s2b, cnt, r2(gn2_w), r2(gn2_b),
                             r2(gn2_ms), batchp, gW1.astype(f32), r2(gb1))

    gate, gmax = _tc_gate(t, st1, st2, r2(gbn_g), r2(gbn_b),
                          gW2.astype(f32), gb2.reshape(1, 1).astype(f32),
                          batchp)
    num, den = _tc_pool(gate, gmax, h, batchp)

    out = _tc_head(num, den, lW1.astype(f32), r2(lb1), r2(lbn1_g), r2(lbn1_b),
                   lW2.astype(f32), r2(lb2), r2(lbn2_g), r2(lbn2_b),
                   lW3.astype(f32), lb3.reshape(1, 1).astype(f32))
    return out
